# Initial kernel scaffold; baseline (speedup 1.0000x reference)
import jax
import jax.numpy as jnp
from jax.experimental import pallas as pl


def kernel(x, edge_index, edge_attr, W1e, b1e, W2e, b2e, W1n, b1n, W2n, b2n):
    raise NotImplementedError("write your pallas kernel here")


# speedup vs baseline: 2.7490x; 2.7490x over previous
import functools

import jax
import jax.numpy as jnp
from jax import lax
from jax.experimental import pallas as pl
from jax.experimental.pallas import tpu as pltpu
from jax.experimental.pallas import tpu_sc as plsc

N = 10000
E = 320000
D = 128
DE = 16

_NC = 2
_NS = 16
_NW = _NC * _NS
_EPW = E // _NW
_CH = 80
_NCH = _EPW // _CH

_mesh = plsc.VectorSubcoreMesh(core_axis_name="c", subcore_axis_name="s")


@functools.partial(
    pl.kernel,
    out_type=jax.ShapeDtypeStruct((E, D), jnp.float32),
    mesh=_mesh,
    scratch_types=[
        pltpu.VMEM((_CH,), jnp.int32),
        pltpu.VMEM((_CH,), jnp.int32),
        pltpu.VMEM((_CH, D), jnp.float32),
        pltpu.VMEM((_CH, D), jnp.float32),
        pltpu.SemaphoreType.DMA,
        pltpu.SemaphoreType.DMA,
    ],
)
def _gather_add(xs_hbm, xd_hbm, src_hbm, dst_hbm, g_hbm,
                idx_s, idx_d, buf_a, buf_b, sem_a, sem_b):
    wid = lax.axis_index("s") * _NC + lax.axis_index("c")
    base = wid * _EPW

    def chunk(ci, carry):
        cbase = base + ci * _CH
        pltpu.sync_copy(src_hbm.at[pl.ds(cbase, _CH)], idx_s)
        pltpu.sync_copy(dst_hbm.at[pl.ds(cbase, _CH)], idx_d)
        ca = pltpu.async_copy(xs_hbm.at[idx_s], buf_a, sem_a)
        cb = pltpu.async_copy(xd_hbm.at[idx_d], buf_b, sem_b)
        ca.wait()
        cb.wait()

        def row(r, c2):
            for j in range(D // 16):
                sl = pl.ds(j * 16, 16)
                buf_a[r, sl] = buf_a[r, sl] + buf_b[r, sl]
            return c2

        lax.fori_loop(0, _CH, row, 0)
        pltpu.sync_copy(buf_a, g_hbm.at[pl.ds(cbase, _CH)])
        return carry

    lax.fori_loop(0, _NCH, chunk, 0)


_NR = N // 2
_NRP = 5008
_NF = DE + 1
_ACC = _NF * _NRP
_CH2 = 128
_TCH = E // _CH2
_CPS = _TCH // _NS

@functools.partial(
    pl.kernel,
    out_type=jax.ShapeDtypeStruct((_NW * _ACC,), jnp.float32),
    mesh=_mesh,
    compiler_params=pltpu.CompilerParams(needs_layout_passes=False),
    scratch_types=[
        pltpu.VMEM((_CH2,), jnp.int32),
        pltpu.VMEM((DE, _CH2), jnp.float32),
        pltpu.VMEM((_ACC,), jnp.float32),
    ],
)
def _scatter_mean(newet_hbm, dst_hbm, acc_out, idx, vbuf, acc):
    cid = lax.axis_index("c")
    sid = lax.axis_index("s")
    wid = sid * _NC + cid
    lo = cid * _NR
    cnt_base = DE * _NRP

    def zvec(r, c2):
        acc[pl.ds(r * 16, 16)] = jnp.zeros((16,), jnp.float32)
        return c2

    lax.fori_loop(0, _ACC // 16, zvec, 0)

    ones = jnp.ones((16,), jnp.float32)
    nch = jnp.where(sid < _TCH - _CPS * _NS, _CPS + 1, _CPS)
    ch0 = sid * _CPS + jnp.minimum(sid, _TCH - _CPS * _NS)

    def chunk(ci, carry):
        cbase = (ch0 + ci) * _CH2
        pltpu.sync_copy(dst_hbm.at[pl.ds(cbase, _CH2)], idx)
        pltpu.sync_copy(newet_hbm.at[:, pl.ds(cbase, _CH2)], vbuf)
        for k in range(_CH2 // 16):
            sl = pl.ds(k * 16, 16)
            idxv = idx[sl] - lo
            mask = (idxv >= 0) & (idxv < _NR)
            plsc.addupdate_scatter(acc, [idxv + cnt_base], ones, mask=mask)
            for f in range(DE):
                plsc.addupdate_scatter(acc, [idxv + f * _NRP], vbuf[f, sl],
                                       mask=mask)
        return carry

    lax.fori_loop(0, nch, chunk, 0)
    pltpu.sync_copy(acc, acc_out.at[pl.ds(wid * _ACC, _ACC)])


def _pre_body(x_ref, ws_ref, wd_ref, xs_ref, xd_ref):
    xb = x_ref[...]
    xs_ref[...] = jnp.dot(xb, ws_ref[...], preferred_element_type=jnp.float32)
    xd_ref[...] = jnp.dot(xb, wd_ref[...], preferred_element_type=jnp.float32)


def _edge_body(g_ref, ea_ref, wea_ref, b1_ref, w2_ref, b2_ref,
               out_ref, out_t_ref):
    t = jnp.dot(ea_ref[...], wea_ref[...], preferred_element_type=jnp.float32)
    h = jnp.maximum(t + g_ref[...] + b1_ref[...], 0.0)
    res = jnp.dot(h, w2_ref[...], preferred_element_type=jnp.float32) + b2_ref[...]
    out_ref[...] = res
    out_t_ref[...] = res.T


def _reduce_body(ap_ref, aggc_ref):
    red = jnp.sum(ap_ref[...], axis=0)
    eye = jnp.eye(_NF, dtype=jnp.float32)
    t0 = jax.lax.dot_general(red[0], eye, (((0,), (0,)), ((), ())),
                             preferred_element_type=jnp.float32)
    t1 = jax.lax.dot_general(red[1], eye, (((0,), (0,)), ((), ())),
                             preferred_element_type=jnp.float32)
    aggc_ref[...] = jnp.concatenate([t0[:_NR], t1[:_NR]], axis=0)


def _node_body(x_ref, ac_ref, w1x_ref, w1a_ref, b1_ref, w2_ref, b2_ref,
               out_ref):
    ac = ac_ref[...]
    a = ac[:, :DE] / jnp.maximum(ac[:, DE:], 1.0)
    t = (jnp.dot(x_ref[...], w1x_ref[...], preferred_element_type=jnp.float32)
         + jnp.dot(a, w1a_ref[...], preferred_element_type=jnp.float32)
         + b1_ref[...])
    h = jnp.maximum(t, 0.0)
    out_ref[...] = (
        jnp.dot(h, w2_ref[...], preferred_element_type=jnp.float32) + b2_ref[...]
    )


def _full(shape):
    return pl.BlockSpec(shape, lambda i: (0,) * len(shape))


@jax.jit
def kernel(x, edge_index, edge_attr, W1e, b1e, W2e, b2e, W1n, b1n, W2n, b2n):
    src = edge_index[0]
    dst = edge_index[1]
    ws, wd, wea = W1e[:D], W1e[D:2 * D], W1e[2 * D:]
    b1e2 = b1e.reshape(1, -1)
    b2e2 = b2e.reshape(1, -1)
    b1n2 = b1n.reshape(1, -1)
    b2n2 = b2n.reshape(1, -1)

    bn = 1000
    xs, xd = pl.pallas_call(
        _pre_body,
        grid=(N // bn,),
        in_specs=[pl.BlockSpec((bn, D), lambda i: (i, 0)),
                  _full((D, D)), _full((D, D))],
        out_specs=[pl.BlockSpec((bn, D), lambda i: (i, 0))] * 2,
        out_shape=[jax.ShapeDtypeStruct((N, D), jnp.float32)] * 2,
    )(x, ws, wd)

    g = _gather_add(xs, xd, src, dst)

    be = 2560
    newe, newet = pl.pallas_call(
        _edge_body,
        grid=(E // be,),
        in_specs=[pl.BlockSpec((be, D), lambda i: (i, 0)),
                  pl.BlockSpec((be, DE), lambda i: (i, 0)),
                  _full((DE, D)), _full((1, D)), _full((D, DE)), _full((1, DE))],
        out_specs=[pl.BlockSpec((be, DE), lambda i: (i, 0)),
                   pl.BlockSpec((DE, be), lambda i: (0, i))],
        out_shape=[jax.ShapeDtypeStruct((E, DE), jnp.float32),
                   jax.ShapeDtypeStruct((DE, E), jnp.float32)],
    )(g, edge_attr, wea, b1e2, W2e, b2e2)

    accp = _scatter_mean(newet, dst)
    aggp = accp.reshape(_NS, _NC, _NF, _NRP)

    aggc = pl.pallas_call(
        _reduce_body,
        grid=(1,),
        in_specs=[_full((_NS, _NC, _NF, _NRP))],
        out_specs=_full((N, _NF)),
        out_shape=jax.ShapeDtypeStruct((N, _NF), jnp.float32),
    )(aggp)

    w1x, w1a = W1n[:D], W1n[D:]
    newx = pl.pallas_call(
        _node_body,
        grid=(N // bn,),
        in_specs=[pl.BlockSpec((bn, D), lambda i: (i, 0)),
                  pl.BlockSpec((bn, _NF), lambda i: (i, 0)),
                  _full((D, D)), _full((DE, D)), _full((1, D)),
                  _full((D, D)), _full((1, D))],
        out_specs=pl.BlockSpec((bn, D), lambda i: (i, 0)),
        out_shape=jax.ShapeDtypeStruct((N, D), jnp.float32),
    )(x, aggc, w1x, w1a, b1n2, W2n, b2n2)

    return (newx, newe)


# speedup vs baseline: 3.3615x; 1.2228x over previous
import functools

import jax
import jax.numpy as jnp
from jax import lax
from jax.experimental import pallas as pl
from jax.experimental.pallas import tpu as pltpu
from jax.experimental.pallas import tpu_sc as plsc

N = 10000
E = 320000
D = 128
DE = 16

_NC = 2
_NS = 16
_NW = _NC * _NS
_EPW = E // _NW
_CH = 80
_NCH = _EPW // _CH

_mesh = plsc.VectorSubcoreMesh(core_axis_name="c", subcore_axis_name="s")


@functools.partial(
    pl.kernel,
    out_type=jax.ShapeDtypeStruct((E, D), jnp.float32),
    mesh=_mesh,
    scratch_types=[
        pltpu.VMEM((_CH,), jnp.int32),
        pltpu.VMEM((_CH,), jnp.int32),
        pltpu.VMEM((_CH,), jnp.int32),
        pltpu.VMEM((_CH,), jnp.int32),
        pltpu.VMEM((_CH, D), jnp.float32),
        pltpu.VMEM((_CH, D), jnp.float32),
        pltpu.VMEM((_CH, D), jnp.float32),
        pltpu.VMEM((_CH, D), jnp.float32),
        pltpu.VMEM((_CH, D), jnp.float32),
        pltpu.VMEM((_CH, D), jnp.float32),
        pltpu.SemaphoreType.DMA,
        pltpu.SemaphoreType.DMA,
        pltpu.SemaphoreType.DMA,
        pltpu.SemaphoreType.DMA,
        pltpu.SemaphoreType.DMA,
        pltpu.SemaphoreType.DMA,
    ],
)
def _gather_add(xs_hbm, xd_hbm, src_hbm, dst_hbm, g_hbm,
                is0, id0, is1, id1, a0, b0, a1, b1, o0, o1,
                sa0, sb0, sa1, sb1, sw0, sw1):
    wid = lax.axis_index("s") * _NC + lax.axis_index("c")
    base = wid * _EPW
    idxs = ((is0, id0), (is1, id1))
    bufs = ((a0, b0), (a1, b1))
    outs = (o0, o1)
    gsem = ((sa0, sb0), (sa1, sb1))
    wsem = (sw0, sw1)

    def start_gathers(ci, slot):
        cb = base + ci * _CH
        pltpu.sync_copy(src_hbm.at[pl.ds(cb, _CH)], idxs[slot][0])
        pltpu.sync_copy(dst_hbm.at[pl.ds(cb, _CH)], idxs[slot][1])
        pltpu.async_copy(xs_hbm.at[idxs[slot][0]], bufs[slot][0],
                         gsem[slot][0])
        pltpu.async_copy(xd_hbm.at[idxs[slot][1]], bufs[slot][1],
                         gsem[slot][1])

    def wait_gathers(slot):
        pltpu.make_async_copy(
            xs_hbm.at[idxs[slot][0]], bufs[slot][0], gsem[slot][0]).wait()
        pltpu.make_async_copy(
            xd_hbm.at[idxs[slot][1]], bufs[slot][1], gsem[slot][1]).wait()

    def add_rows(slot):
        def row(r, c2):
            for j in range(D // 16):
                sl = pl.ds(j * 16, 16)
                outs[slot][r, sl] = bufs[slot][0][r, sl] + bufs[slot][1][r, sl]
            return c2
        lax.fori_loop(0, _CH, row, 0)

    def start_write(ci, slot):
        pltpu.async_copy(outs[slot],
                         g_hbm.at[pl.ds(base + ci * _CH, _CH)], wsem[slot])

    def wait_write(ci, slot):
        pltpu.make_async_copy(
            outs[slot], g_hbm.at[pl.ds(base + ci * _CH, _CH)],
            wsem[slot]).wait()

    start_gathers(0, 0)
    start_gathers(1, 1)
    for half in range(2):
        wait_gathers(half)
        add_rows(half)
        start_write(half, half)
        start_gathers(half + 2, half)

    def pair(p, carry):
        for half in range(2):
            ci = 2 * p + 2 + half
            wait_gathers(half)
            wait_write(ci - 2, half)
            add_rows(half)
            start_write(ci, half)
            start_gathers(ci + 2, half)
        return carry

    lax.fori_loop(0, (_NCH - 3) // 2 - 1, pair, 0)
    ci = _NCH - 3
    wait_gathers(0)
    wait_write(ci - 2, 0)
    add_rows(0)
    start_write(ci, 0)
    start_gathers(_NCH - 1, 0)

    ci = _NCH - 2
    wait_gathers(1)
    wait_write(ci - 2, 1)
    add_rows(1)
    start_write(ci, 1)

    ci = _NCH - 1
    wait_gathers(0)
    wait_write(ci - 2, 0)
    add_rows(0)
    pltpu.sync_copy(outs[0], g_hbm.at[pl.ds(base + ci * _CH, _CH)])
    wait_write(_NCH - 2, 1)


_NR = N // 2
_NRP = 5008
_NF = DE + 1
_ACC = _NF * _NRP
_CH2 = 128
_TCH = E // _CH2
_CPS = _TCH // _NS

@functools.partial(
    pl.kernel,
    out_type=jax.ShapeDtypeStruct((_NW * _ACC,), jnp.float32),
    mesh=_mesh,
    compiler_params=pltpu.CompilerParams(needs_layout_passes=False),
    scratch_types=[
        pltpu.VMEM((_CH2,), jnp.int32),
        pltpu.VMEM((DE, _CH2), jnp.float32),
        pltpu.VMEM((_ACC,), jnp.float32),
    ],
)
def _scatter_mean(newet_hbm, dst_hbm, acc_out, idx, vbuf, acc):
    cid = lax.axis_index("c")
    sid = lax.axis_index("s")
    wid = sid * _NC + cid
    lo = cid * _NR
    cnt_base = DE * _NRP

    def zvec(r, c2):
        acc[pl.ds(r * 16, 16)] = jnp.zeros((16,), jnp.float32)
        return c2

    lax.fori_loop(0, _ACC // 16, zvec, 0)

    ones = jnp.ones((16,), jnp.float32)
    nch = jnp.where(sid < _TCH - _CPS * _NS, _CPS + 1, _CPS)
    ch0 = sid * _CPS + jnp.minimum(sid, _TCH - _CPS * _NS)

    def chunk(ci, carry):
        cbase = (ch0 + ci) * _CH2
        pltpu.sync_copy(dst_hbm.at[pl.ds(cbase, _CH2)], idx)
        pltpu.sync_copy(newet_hbm.at[:, pl.ds(cbase, _CH2)], vbuf)
        for k in range(_CH2 // 16):
            sl = pl.ds(k * 16, 16)
            idxv = idx[sl] - lo
            mask = (idxv >= 0) & (idxv < _NR)
            plsc.addupdate_scatter(acc, [idxv + cnt_base], ones, mask=mask)
            for f in range(DE):
                plsc.addupdate_scatter(acc, [idxv + f * _NRP], vbuf[f, sl],
                                       mask=mask)
        return carry

    lax.fori_loop(0, nch, chunk, 0)
    pltpu.sync_copy(acc, acc_out.at[pl.ds(wid * _ACC, _ACC)])


def _pre_body(x_ref, ws_ref, wd_ref, xs_ref, xd_ref):
    xb = x_ref[...]
    xs_ref[...] = jnp.dot(xb, ws_ref[...], preferred_element_type=jnp.float32)
    xd_ref[...] = jnp.dot(xb, wd_ref[...], preferred_element_type=jnp.float32)


def _edge_body(g_ref, ea_ref, wea_ref, b1_ref, w2_ref, b2_ref,
               out_ref, out_t_ref):
    t = jnp.dot(ea_ref[...], wea_ref[...], preferred_element_type=jnp.float32)
    h = jnp.maximum(t + g_ref[...] + b1_ref[...], 0.0)
    res = jnp.dot(h, w2_ref[...], preferred_element_type=jnp.float32) + b2_ref[...]
    out_ref[...] = res
    out_t_ref[...] = res.T


def _reduce_body(ap_ref, aggc_ref):
    red = jnp.sum(ap_ref[...], axis=0)
    eye = jnp.eye(_NF, dtype=jnp.float32)
    t0 = jax.lax.dot_general(red[0], eye, (((0,), (0,)), ((), ())),
                             preferred_element_type=jnp.float32)
    t1 = jax.lax.dot_general(red[1], eye, (((0,), (0,)), ((), ())),
                             preferred_element_type=jnp.float32)
    aggc_ref[...] = jnp.concatenate([t0[:_NR], t1[:_NR]], axis=0)


def _node_body(x_ref, ac_ref, w1x_ref, w1a_ref, b1_ref, w2_ref, b2_ref,
               out_ref):
    ac = ac_ref[...]
    a = ac[:, :DE] / jnp.maximum(ac[:, DE:], 1.0)
    t = (jnp.dot(x_ref[...], w1x_ref[...], preferred_element_type=jnp.float32)
         + jnp.dot(a, w1a_ref[...], preferred_element_type=jnp.float32)
         + b1_ref[...])
    h = jnp.maximum(t, 0.0)
    out_ref[...] = (
        jnp.dot(h, w2_ref[...], preferred_element_type=jnp.float32) + b2_ref[...]
    )


def _full(shape):
    return pl.BlockSpec(shape, lambda i: (0,) * len(shape))


@jax.jit
def kernel(x, edge_index, edge_attr, W1e, b1e, W2e, b2e, W1n, b1n, W2n, b2n):
    src = edge_index[0]
    dst = edge_index[1]
    ws, wd, wea = W1e[:D], W1e[D:2 * D], W1e[2 * D:]
    b1e2 = b1e.reshape(1, -1)
    b2e2 = b2e.reshape(1, -1)
    b1n2 = b1n.reshape(1, -1)
    b2n2 = b2n.reshape(1, -1)

    bn = 1000
    xs, xd = pl.pallas_call(
        _pre_body,
        grid=(N // bn,),
        in_specs=[pl.BlockSpec((bn, D), lambda i: (i, 0)),
                  _full((D, D)), _full((D, D))],
        out_specs=[pl.BlockSpec((bn, D), lambda i: (i, 0))] * 2,
        out_shape=[jax.ShapeDtypeStruct((N, D), jnp.float32)] * 2,
    )(x, ws, wd)

    g = _gather_add(xs, xd, src, dst)

    be = 2560
    newe, newet = pl.pallas_call(
        _edge_body,
        grid=(E // be,),
        in_specs=[pl.BlockSpec((be, D), lambda i: (i, 0)),
                  pl.BlockSpec((be, DE), lambda i: (i, 0)),
                  _full((DE, D)), _full((1, D)), _full((D, DE)), _full((1, DE))],
        out_specs=[pl.BlockSpec((be, DE), lambda i: (i, 0)),
                   pl.BlockSpec((DE, be), lambda i: (0, i))],
        out_shape=[jax.ShapeDtypeStruct((E, DE), jnp.float32),
                   jax.ShapeDtypeStruct((DE, E), jnp.float32)],
    )(g, edge_attr, wea, b1e2, W2e, b2e2)

    accp = _scatter_mean(newet, dst)
    aggp = accp.reshape(_NS, _NC, _NF, _NRP)

    aggc = pl.pallas_call(
        _reduce_body,
        grid=(1,),
        in_specs=[_full((_NS, _NC, _NF, _NRP))],
        out_specs=_full((N, _NF)),
        out_shape=jax.ShapeDtypeStruct((N, _NF), jnp.float32),
    )(aggp)

    w1x, w1a = W1n[:D], W1n[D:]
    newx = pl.pallas_call(
        _node_body,
        grid=(N // bn,),
        in_specs=[pl.BlockSpec((bn, D), lambda i: (i, 0)),
                  pl.BlockSpec((bn, _NF), lambda i: (i, 0)),
                  _full((D, D)), _full((DE, D)), _full((1, D)),
                  _full((D, D)), _full((1, D))],
        out_specs=pl.BlockSpec((bn, D), lambda i: (i, 0)),
        out_shape=jax.ShapeDtypeStruct((N, D), jnp.float32),
    )(x, aggc, w1x, w1a, b1n2, W2n, b2n2)

    return (newx, newe)


# speedup vs baseline: 4.1447x; 1.2330x over previous
import functools

import jax
import jax.numpy as jnp
from jax import lax
from jax.experimental import pallas as pl
from jax.experimental.pallas import tpu as pltpu
from jax.experimental.pallas import tpu_sc as plsc

N = 10000
E = 320000
D = 128
DE = 16

_NC = 2
_NS = 16
_NW = _NC * _NS
_EPW = E // _NW
_CH = 80
_NCH = _EPW // _CH

_mesh = plsc.VectorSubcoreMesh(core_axis_name="c", subcore_axis_name="s")


@functools.partial(
    pl.kernel,
    out_type=jax.ShapeDtypeStruct((E, D), jnp.float32),
    mesh=_mesh,
    scratch_types=[
        pltpu.VMEM((_CH,), jnp.int32),
        pltpu.VMEM((_CH,), jnp.int32),
        pltpu.VMEM((_CH,), jnp.int32),
        pltpu.VMEM((_CH,), jnp.int32),
        pltpu.VMEM((_CH, D), jnp.float32),
        pltpu.VMEM((_CH, D), jnp.float32),
        pltpu.VMEM((_CH, D), jnp.float32),
        pltpu.VMEM((_CH, D), jnp.float32),
        pltpu.VMEM((_CH, D), jnp.float32),
        pltpu.VMEM((_CH, D), jnp.float32),
        pltpu.SemaphoreType.DMA,
        pltpu.SemaphoreType.DMA,
        pltpu.SemaphoreType.DMA,
        pltpu.SemaphoreType.DMA,
        pltpu.SemaphoreType.DMA,
        pltpu.SemaphoreType.DMA,
    ],
)
def _gather_add(xs_hbm, xd_hbm, src_hbm, dst_hbm, g_hbm,
                is0, id0, is1, id1, a0, b0, a1, b1, o0, o1,
                sa0, sb0, sa1, sb1, sw0, sw1):
    wid = lax.axis_index("s") * _NC + lax.axis_index("c")
    base = wid * _EPW
    idxs = ((is0, id0), (is1, id1))
    bufs = ((a0, b0), (a1, b1))
    outs = (o0, o1)
    gsem = ((sa0, sb0), (sa1, sb1))
    wsem = (sw0, sw1)

    def start_gathers(ci, slot):
        cb = base + ci * _CH
        pltpu.sync_copy(src_hbm.at[pl.ds(cb, _CH)], idxs[slot][0])
        pltpu.sync_copy(dst_hbm.at[pl.ds(cb, _CH)], idxs[slot][1])
        pltpu.async_copy(xs_hbm.at[idxs[slot][0]], bufs[slot][0],
                         gsem[slot][0])
        pltpu.async_copy(xd_hbm.at[idxs[slot][1]], bufs[slot][1],
                         gsem[slot][1])

    def wait_gathers(slot):
        pltpu.make_async_copy(
            xs_hbm.at[idxs[slot][0]], bufs[slot][0], gsem[slot][0]).wait()
        pltpu.make_async_copy(
            xd_hbm.at[idxs[slot][1]], bufs[slot][1], gsem[slot][1]).wait()

    def add_rows(slot):
        def row(r, c2):
            for j in range(D // 16):
                sl = pl.ds(j * 16, 16)
                outs[slot][r, sl] = bufs[slot][0][r, sl] + bufs[slot][1][r, sl]
            return c2
        lax.fori_loop(0, _CH, row, 0)

    def start_write(ci, slot):
        pltpu.async_copy(outs[slot],
                         g_hbm.at[pl.ds(base + ci * _CH, _CH)], wsem[slot])

    def wait_write(ci, slot):
        pltpu.make_async_copy(
            outs[slot], g_hbm.at[pl.ds(base + ci * _CH, _CH)],
            wsem[slot]).wait()

    start_gathers(0, 0)
    start_gathers(1, 1)
    for half in range(2):
        wait_gathers(half)
        add_rows(half)
        start_write(half, half)
        start_gathers(half + 2, half)

    def pair(p, carry):
        for half in range(2):
            ci = 2 * p + 2 + half
            wait_gathers(half)
            wait_write(ci - 2, half)
            add_rows(half)
            start_write(ci, half)
            start_gathers(ci + 2, half)
        return carry

    lax.fori_loop(0, (_NCH - 3) // 2 - 1, pair, 0)
    ci = _NCH - 3
    wait_gathers(0)
    wait_write(ci - 2, 0)
    add_rows(0)
    start_write(ci, 0)
    start_gathers(_NCH - 1, 0)

    ci = _NCH - 2
    wait_gathers(1)
    wait_write(ci - 2, 1)
    add_rows(1)
    start_write(ci, 1)

    ci = _NCH - 1
    wait_gathers(0)
    wait_write(ci - 2, 0)
    add_rows(0)
    pltpu.sync_copy(outs[0], g_hbm.at[pl.ds(base + ci * _CH, _CH)])
    wait_write(_NCH - 2, 1)


_NR = N // 2
_NRP = 5008
_NF = DE + 1
_ACC = _NF * _NRP
_CH2 = 128
_TCH = E // _CH2
_CPS = _TCH // _NS

@functools.partial(
    pl.kernel,
    out_type=jax.ShapeDtypeStruct((_NW * _ACC,), jnp.float32),
    mesh=_mesh,
    compiler_params=pltpu.CompilerParams(needs_layout_passes=False),
    scratch_types=[
        pltpu.VMEM((_CH2,), jnp.int32),
        pltpu.VMEM((_CH2,), jnp.int32),
        pltpu.VMEM((DE, _CH2), jnp.float32),
        pltpu.VMEM((DE, _CH2), jnp.float32),
        pltpu.VMEM((_ACC,), jnp.float32),
        pltpu.SemaphoreType.DMA,
        pltpu.SemaphoreType.DMA,
        pltpu.SemaphoreType.DMA,
        pltpu.SemaphoreType.DMA,
    ],
)
def _scatter_mean(newet_hbm, dst_hbm, acc_out,
                  ix0, ix1, vb0, vb1, acc, si0, si1, sv0, sv1):
    cid = lax.axis_index("c")
    sid = lax.axis_index("s")
    wid = sid * _NC + cid
    lo = cid * _NR
    cnt_base = DE * _NRP
    idxs = (ix0, ix1)
    vbufs = (vb0, vb1)
    isem = (si0, si1)
    vsem = (sv0, sv1)

    def zvec(r, c2):
        acc[pl.ds(r * 16, 16)] = jnp.zeros((16,), jnp.float32)
        return c2

    lax.fori_loop(0, _ACC // 16, zvec, 0)

    ones = jnp.ones((16,), jnp.float32)

    def start_fetch(ci, slot):
        cbase = (sid * _CPS + ci) * _CH2
        pltpu.async_copy(dst_hbm.at[pl.ds(cbase, _CH2)], idxs[slot],
                         isem[slot])
        pltpu.async_copy(newet_hbm.at[:, pl.ds(cbase, _CH2)], vbufs[slot],
                         vsem[slot])

    def wait_fetch(ci, slot):
        cbase = (sid * _CPS + ci) * _CH2
        pltpu.make_async_copy(dst_hbm.at[pl.ds(cbase, _CH2)], idxs[slot],
                              isem[slot]).wait()
        pltpu.make_async_copy(newet_hbm.at[:, pl.ds(cbase, _CH2)],
                              vbufs[slot], vsem[slot]).wait()

    def scatter(slot):
        idx = idxs[slot]
        vbuf = vbufs[slot]
        for k in range(_CH2 // 16):
            sl = pl.ds(k * 16, 16)
            idxv = idx[sl] - lo
            mask = (idxv >= 0) & (idxv < _NR)
            plsc.addupdate_scatter(acc, [idxv + cnt_base], ones, mask=mask)
            for f in range(DE):
                plsc.addupdate_scatter(acc, [idxv + f * _NRP], vbuf[f, sl],
                                       mask=mask)

    start_fetch(0, 0)
    start_fetch(1, 1)

    def pair(p, carry):
        for half in range(2):
            ci = 2 * p + half
            wait_fetch(ci, half)
            scatter(half)
            start_fetch(ci + 2, half)
        return carry

    lax.fori_loop(0, (_CPS - 2) // 2, pair, 0)
    wait_fetch(_CPS - 2, 0)
    scatter(0)
    wait_fetch(_CPS - 1, 1)
    scatter(1)

    @pl.when(sid < _TCH - _CPS * _NS)
    def _():
        cbase = (_CPS * _NS + sid) * _CH2
        pltpu.sync_copy(dst_hbm.at[pl.ds(cbase, _CH2)], idxs[0])
        pltpu.sync_copy(newet_hbm.at[:, pl.ds(cbase, _CH2)], vbufs[0])
        scatter(0)

    pltpu.sync_copy(acc, acc_out.at[pl.ds(wid * _ACC, _ACC)])


def _pre_body(x_ref, ws_ref, wd_ref, xs_ref, xd_ref):
    xb = x_ref[...]
    xs_ref[...] = jnp.dot(xb, ws_ref[...], preferred_element_type=jnp.float32)
    xd_ref[...] = jnp.dot(xb, wd_ref[...], preferred_element_type=jnp.float32)


def _edge_body(g_ref, ea_ref, wea_ref, b1_ref, w2_ref, b2_ref,
               out_ref, out_t_ref):
    t = jnp.dot(ea_ref[...], wea_ref[...], preferred_element_type=jnp.float32)
    h = jnp.maximum(t + g_ref[...] + b1_ref[...], 0.0)
    res = jnp.dot(h, w2_ref[...], preferred_element_type=jnp.float32) + b2_ref[...]
    out_ref[...] = res
    out_t_ref[...] = res.T


def _reduce_body(ap_ref, aggc_ref):
    red = jnp.sum(ap_ref[...], axis=0)
    eye = jnp.eye(_NF, dtype=jnp.float32)
    t0 = jax.lax.dot_general(red[0], eye, (((0,), (0,)), ((), ())),
                             preferred_element_type=jnp.float32)
    t1 = jax.lax.dot_general(red[1], eye, (((0,), (0,)), ((), ())),
                             preferred_element_type=jnp.float32)
    aggc_ref[...] = jnp.concatenate([t0[:_NR], t1[:_NR]], axis=0)


def _node_body(x_ref, ac_ref, w1x_ref, w1a_ref, b1_ref, w2_ref, b2_ref,
               out_ref):
    ac = ac_ref[...]
    a = ac[:, :DE] / jnp.maximum(ac[:, DE:], 1.0)
    t = (jnp.dot(x_ref[...], w1x_ref[...], preferred_element_type=jnp.float32)
         + jnp.dot(a, w1a_ref[...], preferred_element_type=jnp.float32)
         + b1_ref[...])
    h = jnp.maximum(t, 0.0)
    out_ref[...] = (
        jnp.dot(h, w2_ref[...], preferred_element_type=jnp.float32) + b2_ref[...]
    )


def _full(shape):
    return pl.BlockSpec(shape, lambda i: (0,) * len(shape))


@jax.jit
def kernel(x, edge_index, edge_attr, W1e, b1e, W2e, b2e, W1n, b1n, W2n, b2n):
    src = edge_index[0]
    dst = edge_index[1]
    ws, wd, wea = W1e[:D], W1e[D:2 * D], W1e[2 * D:]
    b1e2 = b1e.reshape(1, -1)
    b2e2 = b2e.reshape(1, -1)
    b1n2 = b1n.reshape(1, -1)
    b2n2 = b2n.reshape(1, -1)

    bn = 1000
    xs, xd = pl.pallas_call(
        _pre_body,
        grid=(N // bn,),
        in_specs=[pl.BlockSpec((bn, D), lambda i: (i, 0)),
                  _full((D, D)), _full((D, D))],
        out_specs=[pl.BlockSpec((bn, D), lambda i: (i, 0))] * 2,
        out_shape=[jax.ShapeDtypeStruct((N, D), jnp.float32)] * 2,
    )(x, ws, wd)

    g = _gather_add(xs, xd, src, dst)

    be = 2560
    newe, newet = pl.pallas_call(
        _edge_body,
        grid=(E // be,),
        in_specs=[pl.BlockSpec((be, D), lambda i: (i, 0)),
                  pl.BlockSpec((be, DE), lambda i: (i, 0)),
                  _full((DE, D)), _full((1, D)), _full((D, DE)), _full((1, DE))],
        out_specs=[pl.BlockSpec((be, DE), lambda i: (i, 0)),
                   pl.BlockSpec((DE, be), lambda i: (0, i))],
        out_shape=[jax.ShapeDtypeStruct((E, DE), jnp.float32),
                   jax.ShapeDtypeStruct((DE, E), jnp.float32)],
    )(g, edge_attr, wea, b1e2, W2e, b2e2)

    accp = _scatter_mean(newet, dst)
    aggp = accp.reshape(_NS, _NC, _NF, _NRP)

    aggc = pl.pallas_call(
        _reduce_body,
        grid=(1,),
        in_specs=[_full((_NS, _NC, _NF, _NRP))],
        out_specs=_full((N, _NF)),
        out_shape=jax.ShapeDtypeStruct((N, _NF), jnp.float32),
    )(aggp)

    w1x, w1a = W1n[:D], W1n[D:]
    newx = pl.pallas_call(
        _node_body,
        grid=(N // bn,),
        in_specs=[pl.BlockSpec((bn, D), lambda i: (i, 0)),
                  pl.BlockSpec((bn, _NF), lambda i: (i, 0)),
                  _full((D, D)), _full((DE, D)), _full((1, D)),
                  _full((D, D)), _full((1, D))],
        out_specs=pl.BlockSpec((bn, D), lambda i: (i, 0)),
        out_shape=jax.ShapeDtypeStruct((N, D), jnp.float32),
    )(x, aggc, w1x, w1a, b1n2, W2n, b2n2)

    return (newx, newe)


# speedup vs baseline: 4.2763x; 1.0317x over previous
import functools

import jax
import jax.numpy as jnp
from jax import lax
from jax.experimental import pallas as pl
from jax.experimental.pallas import tpu as pltpu
from jax.experimental.pallas import tpu_sc as plsc

N = 10000
E = 320000
D = 128
DE = 16

_NC = 2
_NS = 16
_NW = _NC * _NS
_EPW = E // _NW
_CH = 80
_NCH = _EPW // _CH

_mesh = plsc.VectorSubcoreMesh(core_axis_name="c", subcore_axis_name="s")


@functools.partial(
    pl.kernel,
    out_type=jax.ShapeDtypeStruct((E, D), jnp.float32),
    mesh=_mesh,
    scratch_types=[
        pltpu.VMEM((_CH,), jnp.int32),
        pltpu.VMEM((_CH,), jnp.int32),
        pltpu.VMEM((_CH,), jnp.int32),
        pltpu.VMEM((_CH,), jnp.int32),
        pltpu.VMEM((_CH,), jnp.int32),
        pltpu.VMEM((_CH,), jnp.int32),
        pltpu.VMEM((_CH, D), jnp.float32),
        pltpu.VMEM((_CH, D), jnp.float32),
        pltpu.VMEM((_CH, D), jnp.float32),
        pltpu.SemaphoreType.DMA,
        pltpu.SemaphoreType.DMA,
        pltpu.SemaphoreType.DMA,
        pltpu.SemaphoreType.DMA,
        pltpu.SemaphoreType.DMA,
        pltpu.SemaphoreType.DMA,
        pltpu.SemaphoreType.DMA,
        pltpu.SemaphoreType.DMA,
        pltpu.SemaphoreType.DMA,
    ],
)
def _gather_add(xs_hbm, xd_hbm, src_hbm, dst_hbm, g_hbm,
                is0, is1, is2, id0, id1, id2, b0, b1, b2,
                sf0, sf1, sf2, sa0, sa1, sa2, sw0, sw1, sw2):
    wid = lax.axis_index("s") * _NC + lax.axis_index("c")
    base = wid * _EPW
    iss = (is0, is1, is2)
    ids = (id0, id1, id2)
    bufs = (b0, b1, b2)
    fsem = (sf0, sf1, sf2)
    asem = (sa0, sa1, sa2)
    wsem = (sw0, sw1, sw2)

    def start_if(c, sl):
        cb = base + c * _CH
        pltpu.async_copy(src_hbm.at[pl.ds(cb, _CH)], iss[sl], fsem[sl])
        pltpu.async_copy(dst_hbm.at[pl.ds(cb, _CH)], ids[sl], fsem[sl])

    def wait_if(c, sl):
        cb = base + c * _CH
        pltpu.make_async_copy(src_hbm.at[pl.ds(cb, _CH)], iss[sl],
                              fsem[sl]).wait()
        pltpu.make_async_copy(dst_hbm.at[pl.ds(cb, _CH)], ids[sl],
                              fsem[sl]).wait()

    def start_a(sl):
        pltpu.async_copy(xs_hbm.at[iss[sl]], bufs[sl], asem[sl])

    def wait_a(sl):
        pltpu.make_async_copy(xs_hbm.at[iss[sl]], bufs[sl], asem[sl]).wait()

    def start_b(sl):
        pltpu.async_copy(xd_hbm.at[ids[sl]], bufs[sl], asem[sl], add=True)

    def wait_b(sl):
        pltpu.make_async_copy(xd_hbm.at[ids[sl]], bufs[sl], asem[sl]).wait()

    def start_w(c, sl):
        pltpu.async_copy(bufs[sl], g_hbm.at[pl.ds(base + c * _CH, _CH)],
                         wsem[sl])

    def wait_w(c, sl):
        pltpu.make_async_copy(bufs[sl], g_hbm.at[pl.ds(base + c * _CH, _CH)],
                              wsem[sl]).wait()

    def body(v, sv, with_if=True, with_ww=True):
        s_prev = (sv + 2) % 3
        s_next = (sv + 1) % 3
        wait_b(s_prev)
        start_w(v - 1, s_prev)
        if with_if:
            start_if(v + 2, s_prev)
        wait_if(v + 1, s_next)
        if with_ww:
            wait_w(v - 2, s_next)
        start_a(s_next)
        wait_a(sv)
        start_b(sv)

    start_if(0, 0)
    start_if(1, 1)
    wait_if(0, 0)
    start_a(0)
    start_if(2, 2)
    wait_if(1, 1)
    start_a(1)
    wait_a(0)
    start_b(0)
    body(1, 1, with_ww=False)

    def steady(p, carry):
        for j in range(3):
            body(3 * p + 2 + j, (2 + j) % 3)
        return carry

    lax.fori_loop(0, (_NCH - 5) // 3, steady, 0)
    body(_NCH - 3, 2)
    body(_NCH - 2, 0, with_if=False)
    wait_b(0)
    start_w(_NCH - 2, 0)
    wait_w(_NCH - 3, 2)
    wait_a(1)
    start_b(1)
    wait_b(1)
    pltpu.sync_copy(bufs[1], g_hbm.at[pl.ds(base + (_NCH - 1) * _CH, _CH)])
    wait_w(_NCH - 2, 0)


_NR = N // 2
_NRP = 5008
_NF = DE + 1
_ACC = _NF * _NRP
_CH2 = 128
_TCH = E // _CH2
_CPS = _TCH // _NS

@functools.partial(
    pl.kernel,
    out_type=jax.ShapeDtypeStruct((_NW * _ACC,), jnp.float32),
    mesh=_mesh,
    compiler_params=pltpu.CompilerParams(needs_layout_passes=False),
    scratch_types=[
        pltpu.VMEM((_CH2,), jnp.int32),
        pltpu.VMEM((_CH2,), jnp.int32),
        pltpu.VMEM((DE, _CH2), jnp.float32),
        pltpu.VMEM((DE, _CH2), jnp.float32),
        pltpu.VMEM((_ACC,), jnp.float32),
        pltpu.SemaphoreType.DMA,
        pltpu.SemaphoreType.DMA,
        pltpu.SemaphoreType.DMA,
        pltpu.SemaphoreType.DMA,
    ],
)
def _scatter_mean(newet_hbm, dst_hbm, acc_out,
                  ix0, ix1, vb0, vb1, acc, si0, si1, sv0, sv1):
    cid = lax.axis_index("c")
    sid = lax.axis_index("s")
    wid = sid * _NC + cid
    lo = cid * _NR
    cnt_base = DE * _NRP
    idxs = (ix0, ix1)
    vbufs = (vb0, vb1)
    isem = (si0, si1)
    vsem = (sv0, sv1)

    def zvec(r, c2):
        acc[pl.ds(r * 16, 16)] = jnp.zeros((16,), jnp.float32)
        return c2

    lax.fori_loop(0, _ACC // 16, zvec, 0)

    ones = jnp.ones((16,), jnp.float32)

    def start_fetch(ci, slot):
        cbase = (sid * _CPS + ci) * _CH2
        pltpu.async_copy(dst_hbm.at[pl.ds(cbase, _CH2)], idxs[slot],
                         isem[slot])
        pltpu.async_copy(newet_hbm.at[:, pl.ds(cbase, _CH2)], vbufs[slot],
                         vsem[slot])

    def wait_fetch(ci, slot):
        cbase = (sid * _CPS + ci) * _CH2
        pltpu.make_async_copy(dst_hbm.at[pl.ds(cbase, _CH2)], idxs[slot],
                              isem[slot]).wait()
        pltpu.make_async_copy(newet_hbm.at[:, pl.ds(cbase, _CH2)],
                              vbufs[slot], vsem[slot]).wait()

    def scatter(slot):
        idx = idxs[slot]
        vbuf = vbufs[slot]
        for k in range(_CH2 // 16):
            sl = pl.ds(k * 16, 16)
            idxv = idx[sl] - lo
            mask = (idxv >= 0) & (idxv < _NR)
            plsc.addupdate_scatter(acc, [idxv + cnt_base], ones, mask=mask)
            for f in range(DE):
                plsc.addupdate_scatter(acc, [idxv + f * _NRP], vbuf[f, sl],
                                       mask=mask)

    start_fetch(0, 0)
    start_fetch(1, 1)

    def pair(p, carry):
        for half in range(2):
            ci = 2 * p + half
            wait_fetch(ci, half)
            scatter(half)
            start_fetch(ci + 2, half)
        return carry

    lax.fori_loop(0, (_CPS - 2) // 2, pair, 0)
    wait_fetch(_CPS - 2, 0)
    scatter(0)
    wait_fetch(_CPS - 1, 1)
    scatter(1)

    @pl.when(sid < _TCH - _CPS * _NS)
    def _():
        cbase = (_CPS * _NS + sid) * _CH2
        pltpu.sync_copy(dst_hbm.at[pl.ds(cbase, _CH2)], idxs[0])
        pltpu.sync_copy(newet_hbm.at[:, pl.ds(cbase, _CH2)], vbufs[0])
        scatter(0)

    pltpu.sync_copy(acc, acc_out.at[pl.ds(wid * _ACC, _ACC)])


def _pre_body(x_ref, ws_ref, wd_ref, xs_ref, xd_ref):
    xb = x_ref[...]
    xs_ref[...] = jnp.dot(xb, ws_ref[...], preferred_element_type=jnp.float32)
    xd_ref[...] = jnp.dot(xb, wd_ref[...], preferred_element_type=jnp.float32)


def _edge_body(g_ref, ea_ref, wea_ref, b1_ref, w2_ref, b2_ref,
               out_ref, out_t_ref):
    t = jnp.dot(ea_ref[...], wea_ref[...], preferred_element_type=jnp.float32)
    h = jnp.maximum(t + g_ref[...] + b1_ref[...], 0.0)
    res = jnp.dot(h, w2_ref[...], preferred_element_type=jnp.float32) + b2_ref[...]
    out_ref[...] = res
    out_t_ref[...] = res.T


def _reduce_body(ap_ref, aggc_ref):
    red = jnp.sum(ap_ref[...], axis=0)
    eye = jnp.eye(_NF, dtype=jnp.float32)
    t0 = jax.lax.dot_general(red[0], eye, (((0,), (0,)), ((), ())),
                             preferred_element_type=jnp.float32)
    t1 = jax.lax.dot_general(red[1], eye, (((0,), (0,)), ((), ())),
                             preferred_element_type=jnp.float32)
    aggc_ref[...] = jnp.concatenate([t0[:_NR], t1[:_NR]], axis=0)


def _node_body(x_ref, ac_ref, w1x_ref, w1a_ref, b1_ref, w2_ref, b2_ref,
               out_ref):
    ac = ac_ref[...]
    a = ac[:, :DE] / jnp.maximum(ac[:, DE:], 1.0)
    t = (jnp.dot(x_ref[...], w1x_ref[...], preferred_element_type=jnp.float32)
         + jnp.dot(a, w1a_ref[...], preferred_element_type=jnp.float32)
         + b1_ref[...])
    h = jnp.maximum(t, 0.0)
    out_ref[...] = (
        jnp.dot(h, w2_ref[...], preferred_element_type=jnp.float32) + b2_ref[...]
    )


def _full(shape):
    return pl.BlockSpec(shape, lambda i: (0,) * len(shape))


@jax.jit
def kernel(x, edge_index, edge_attr, W1e, b1e, W2e, b2e, W1n, b1n, W2n, b2n):
    src = edge_index[0]
    dst = edge_index[1]
    ws, wd, wea = W1e[:D], W1e[D:2 * D], W1e[2 * D:]
    b1e2 = b1e.reshape(1, -1)
    b2e2 = b2e.reshape(1, -1)
    b1n2 = b1n.reshape(1, -1)
    b2n2 = b2n.reshape(1, -1)

    bn = 1000
    xs, xd = pl.pallas_call(
        _pre_body,
        grid=(N // bn,),
        in_specs=[pl.BlockSpec((bn, D), lambda i: (i, 0)),
                  _full((D, D)), _full((D, D))],
        out_specs=[pl.BlockSpec((bn, D), lambda i: (i, 0))] * 2,
        out_shape=[jax.ShapeDtypeStruct((N, D), jnp.float32)] * 2,
    )(x, ws, wd)

    g = _gather_add(xs, xd, src, dst)

    be = 2560
    newe, newet = pl.pallas_call(
        _edge_body,
        grid=(E // be,),
        in_specs=[pl.BlockSpec((be, D), lambda i: (i, 0)),
                  pl.BlockSpec((be, DE), lambda i: (i, 0)),
                  _full((DE, D)), _full((1, D)), _full((D, DE)), _full((1, DE))],
        out_specs=[pl.BlockSpec((be, DE), lambda i: (i, 0)),
                   pl.BlockSpec((DE, be), lambda i: (0, i))],
        out_shape=[jax.ShapeDtypeStruct((E, DE), jnp.float32),
                   jax.ShapeDtypeStruct((DE, E), jnp.float32)],
    )(g, edge_attr, wea, b1e2, W2e, b2e2)

    accp = _scatter_mean(newet, dst)
    aggp = accp.reshape(_NS, _NC, _NF, _NRP)

    aggc = pl.pallas_call(
        _reduce_body,
        grid=(1,),
        in_specs=[_full((_NS, _NC, _NF, _NRP))],
        out_specs=_full((N, _NF)),
        out_shape=jax.ShapeDtypeStruct((N, _NF), jnp.float32),
    )(aggp)

    w1x, w1a = W1n[:D], W1n[D:]
    newx = pl.pallas_call(
        _node_body,
        grid=(N // bn,),
        in_specs=[pl.BlockSpec((bn, D), lambda i: (i, 0)),
                  pl.BlockSpec((bn, _NF), lambda i: (i, 0)),
                  _full((D, D)), _full((DE, D)), _full((1, D)),
                  _full((D, D)), _full((1, D))],
        out_specs=pl.BlockSpec((bn, D), lambda i: (i, 0)),
        out_shape=jax.ShapeDtypeStruct((N, D), jnp.float32),
    )(x, aggc, w1x, w1a, b1n2, W2n, b2n2)

    return (newx, newe)
